# baseline (speedup 1.0000x reference)
import jax
import jax.numpy as jnp
from jax.experimental import pallas as pl

B, N, D_IN, HIDDEN = 4096, 50, 64, 128
T = B * N
ROWS = 2048
MR = 2 * ROWS // 128
GRID = T // (2 * ROWS)


def _tok_kernel(x_ref, m_ref, e_ref, fl_ref, fr_ref,
                w1_ref, w2_ref, b1_ref, b2_ref, o_ref):
    w = (w1_ref[...] + w2_ref[...]).astype(jnp.bfloat16)
    z = jnp.zeros((D_IN, HIDDEN), jnp.bfloat16)
    wbd = jnp.concatenate(
        [jnp.concatenate([w, z], axis=1),
         jnp.concatenate([z, w], axis=1)], axis=0)
    bb = b1_ref[...] + b2_ref[...]
    bbd = jnp.concatenate([bb, bb], axis=1)

    acc = jax.lax.dot_general(
        x_ref[...].astype(jnp.bfloat16), wbd,
        dimension_numbers=(((1,), (0,)), ((), ())),
        preferred_element_type=jnp.float32,
    )

    g = jax.lax.dot_general(
        e_ref[...], m_ref[...],
        dimension_numbers=(((1,), (0,)), ((), ())),
        preferred_element_type=jnp.float32,
    ).astype(jnp.bfloat16)
    ones = jnp.ones((128, HIDDEN), jnp.bfloat16)
    hl = jax.lax.dot_general(
        g * fl_ref[...], ones,
        dimension_numbers=(((1,), (0,)), ((), ())),
        preferred_element_type=jnp.float32,
    )
    hr = jax.lax.dot_general(
        g * fr_ref[...], ones,
        dimension_numbers=(((1,), (0,)), ((), ())),
        preferred_element_type=jnp.float32,
    )
    h = jnp.concatenate([hl, hr], axis=1)

    o_ref[...] = (acc + bbd) * h


def kernel(x_feats, feats_masks, W1, b1, W2, b2):
    x2 = x_feats.reshape(T // 2, 2 * D_IN)
    m2 = feats_masks.astype(jnp.bfloat16).reshape(T // 128, 128)
    b1r = b1.reshape(1, HIDDEN)
    b2r = b2.reshape(1, HIDDEN)

    r_idx = jnp.arange(ROWS, dtype=jnp.int32)
    q_idx = jnp.arange(128, dtype=jnp.int32)
    e_mat = (r_idx[:, None] // 64 == jnp.arange(MR, dtype=jnp.int32)[None, :]
             ).astype(jnp.bfloat16)
    fl_mat = (q_idx[None, :] == (2 * r_idx[:, None]) % 128
              ).astype(jnp.bfloat16)
    fr_mat = (q_idx[None, :] == (2 * r_idx[:, None] + 1) % 128
              ).astype(jnp.bfloat16)

    out = pl.pallas_call(
        _tok_kernel,
        grid=(GRID,),
        in_specs=[
            pl.BlockSpec((ROWS, 2 * D_IN), lambda i: (i, 0)),
            pl.BlockSpec((MR, 128), lambda i: (i, 0)),
            pl.BlockSpec((ROWS, MR), lambda i: (0, 0)),
            pl.BlockSpec((ROWS, 128), lambda i: (0, 0)),
            pl.BlockSpec((ROWS, 128), lambda i: (0, 0)),
            pl.BlockSpec((D_IN, HIDDEN), lambda i: (0, 0)),
            pl.BlockSpec((D_IN, HIDDEN), lambda i: (0, 0)),
            pl.BlockSpec((1, HIDDEN), lambda i: (0, 0)),
            pl.BlockSpec((1, HIDDEN), lambda i: (0, 0)),
        ],
        out_specs=pl.BlockSpec((ROWS, 2 * HIDDEN), lambda i: (i, 0)),
        out_shape=jax.ShapeDtypeStruct((T // 2, 2 * HIDDEN), jnp.float32),
    )(x2, m2, e_mat, fl_mat, fr_mat, W1, W2, b1r, b2r)
    return out.reshape(B, N, HIDDEN)


# speedup vs baseline: 5.7198x; 5.7198x over previous
import jax
import jax.numpy as jnp
from jax.experimental import pallas as pl

B, N, D_IN, HIDDEN = 4096, 50, 64, 128
NS = 2


def _tok_kernel(x_ref, m_ref, w1_ref, w2_ref, b1_ref, b2_ref, o_ref):
    w = (w1_ref[...] + w2_ref[...]).astype(jnp.bfloat16)
    b = b1_ref[...] + b2_ref[...]
    i = pl.program_id(0)
    for k in range(NS):
        slab = x_ref[k]
        mrow = m_ref[pl.ds(i * NS + k, 1), :]
        aug = jnp.concatenate([slab, mrow], axis=0)
        augt = jnp.transpose(aug, (1, 0))
        mcol = augt[:, D_IN:D_IN + 1]
        xm = augt[:, :D_IN] * mcol
        acc = jax.lax.dot_general(
            xm.astype(jnp.bfloat16), w,
            dimension_numbers=(((1,), (0,)), ((), ())),
            preferred_element_type=jnp.float32,
        )
        o_ref[k] = acc + mcol * b


def kernel(x_feats, feats_masks, W1, b1, W2, b2):
    xt = jnp.transpose(x_feats, (1, 2, 0))
    mt = jnp.transpose(feats_masks, (1, 0)).astype(jnp.float32)
    b1r = b1.reshape(1, HIDDEN)
    b2r = b2.reshape(1, HIDDEN)

    out = pl.pallas_call(
        _tok_kernel,
        grid=(N // NS,),
        in_specs=[
            pl.BlockSpec((NS, D_IN, B), lambda i: (i, 0, 0)),
            pl.BlockSpec((N, B), lambda i: (0, 0)),
            pl.BlockSpec((D_IN, HIDDEN), lambda i: (0, 0)),
            pl.BlockSpec((D_IN, HIDDEN), lambda i: (0, 0)),
            pl.BlockSpec((1, HIDDEN), lambda i: (0, 0)),
            pl.BlockSpec((1, HIDDEN), lambda i: (0, 0)),
        ],
        out_specs=pl.BlockSpec((NS, B, HIDDEN), lambda i: (i, 0, 0)),
        out_shape=jax.ShapeDtypeStruct((N, B, HIDDEN), jnp.float32),
    )(xt, mt, W1, W2, b1r, b2r)
    return jnp.transpose(out, (1, 0, 2))


# speedup vs baseline: 7.3719x; 1.2889x over previous
import jax
import jax.numpy as jnp
from jax.experimental import pallas as pl

B, N, D_IN, HIDDEN = 4096, 50, 64, 128
NS = 2


def _tok_kernel(x_ref, m_ref, w1_ref, w2_ref, b1_ref, b2_ref, o_ref):
    w = (w1_ref[...] + w2_ref[...]).astype(jnp.bfloat16)
    b = (b1_ref[...] + b2_ref[...]).astype(jnp.bfloat16)
    waug = jnp.concatenate([w, b], axis=0)
    i = pl.program_id(0)
    for k in range(NS):
        slab = x_ref[k]
        mrow = m_ref[pl.ds(i * NS + k, 1), :]
        aug = jnp.concatenate([slab, mrow], axis=0)
        aug_m = aug * mrow
        acc = jax.lax.dot_general(
            aug_m.astype(jnp.bfloat16), waug,
            dimension_numbers=(((0,), (0,)), ((), ())),
            preferred_element_type=jnp.float32,
        )
        o_ref[k] = acc


def kernel(x_feats, feats_masks, W1, b1, W2, b2):
    xt = jnp.transpose(x_feats, (1, 2, 0))
    mt = jnp.transpose(feats_masks, (1, 0)).astype(jnp.float32)
    b1r = b1.reshape(1, HIDDEN)
    b2r = b2.reshape(1, HIDDEN)

    out = pl.pallas_call(
        _tok_kernel,
        grid=(N // NS,),
        in_specs=[
            pl.BlockSpec((NS, D_IN, B), lambda i: (i, 0, 0)),
            pl.BlockSpec((N, B), lambda i: (0, 0)),
            pl.BlockSpec((D_IN, HIDDEN), lambda i: (0, 0)),
            pl.BlockSpec((D_IN, HIDDEN), lambda i: (0, 0)),
            pl.BlockSpec((1, HIDDEN), lambda i: (0, 0)),
            pl.BlockSpec((1, HIDDEN), lambda i: (0, 0)),
        ],
        out_specs=pl.BlockSpec((NS, B, HIDDEN), lambda i: (i, 0, 0)),
        out_shape=jax.ShapeDtypeStruct((N, B, HIDDEN), jnp.float32),
    )(xt, mt, W1, W2, b1r, b2r)
    return jnp.transpose(out, (1, 0, 2))


# speedup vs baseline: 8.0370x; 1.0902x over previous
import jax
import jax.numpy as jnp
from jax.experimental import pallas as pl

B, N, D_IN, HIDDEN = 4096, 50, 64, 128
NS = 5


def _tok_kernel(x_ref, m_ref, w1_ref, w2_ref, b1_ref, b2_ref, o_ref):
    w = (w1_ref[...] + w2_ref[...]).astype(jnp.bfloat16)
    b = (b1_ref[...] + b2_ref[...]).astype(jnp.bfloat16)
    waug = jnp.concatenate([w, b], axis=0)
    i = pl.program_id(0)
    for k in range(NS):
        slab = x_ref[k]
        mrow = m_ref[pl.ds(i * NS + k, 1), :]
        aug = jnp.concatenate([slab, mrow], axis=0)
        aug_m = aug * mrow
        acc = jax.lax.dot_general(
            aug_m.astype(jnp.bfloat16), waug,
            dimension_numbers=(((0,), (0,)), ((), ())),
            preferred_element_type=jnp.float32,
        )
        o_ref[k] = acc


def kernel(x_feats, feats_masks, W1, b1, W2, b2):
    xt = jnp.transpose(x_feats, (1, 2, 0))
    mt = jnp.transpose(feats_masks, (1, 0)).astype(jnp.float32)
    b1r = b1.reshape(1, HIDDEN)
    b2r = b2.reshape(1, HIDDEN)

    out = pl.pallas_call(
        _tok_kernel,
        grid=(N // NS,),
        in_specs=[
            pl.BlockSpec((NS, D_IN, B), lambda i: (i, 0, 0)),
            pl.BlockSpec((N, B), lambda i: (0, 0)),
            pl.BlockSpec((D_IN, HIDDEN), lambda i: (0, 0)),
            pl.BlockSpec((D_IN, HIDDEN), lambda i: (0, 0)),
            pl.BlockSpec((1, HIDDEN), lambda i: (0, 0)),
            pl.BlockSpec((1, HIDDEN), lambda i: (0, 0)),
        ],
        out_specs=pl.BlockSpec((NS, B, HIDDEN), lambda i: (i, 0, 0)),
        out_shape=jax.ShapeDtypeStruct((N, B, HIDDEN), jnp.float32),
    )(xt, mt, W1, W2, b1r, b2r)
    return jnp.transpose(out, (1, 0, 2))
